# baseline (speedup 1.0000x reference)
import jax
import jax.numpy as jnp
from jax import lax
from jax.experimental import pallas as pl
from jax.experimental.pallas import tpu as pltpu
from jax.experimental.pallas import tpu_sc as plsc

N_VOTERS = 32
N_COLS = 1_000_000
LANES = 16
NUM_WORKERS = 32
CHUNK = 1600
NCHUNKS = N_COLS // CHUNK
MAX_CHUNKS_PER_WORKER = -(-NCHUNKS // NUM_WORKERS)


def _body(in_hbm, out_hbm, buf, out_buf):
    c = lax.axis_index("c")
    s = lax.axis_index("s")
    wid = s * 2 + c

    def do_chunk(chunk_id):
        base = chunk_id * CHUNK
        pltpu.sync_copy(in_hbm.at[:, pl.ds(base, CHUNK)], buf)

        def col_group(j, _):
            off = j * LANES
            acc = buf[0, pl.ds(off, LANES)]
            for i in range(1, N_VOTERS):
                acc = acc + buf[i, pl.ds(off, LANES)]
            out_buf[pl.ds(off, LANES)] = jnp.where(
                acc > N_VOTERS // 2, 1, 0
            ).astype(jnp.int32)
            return 0

        lax.fori_loop(0, CHUNK // LANES, col_group, 0)
        pltpu.sync_copy(out_buf, out_hbm.at[pl.ds(base, CHUNK)])

    for k in range(MAX_CHUNKS_PER_WORKER):
        chunk_id = wid + k * NUM_WORKERS

        @pl.when(chunk_id < NCHUNKS)
        def _():
            do_chunk(chunk_id)


@jax.jit
def _vote(inputs):
    k = pl.kernel(
        _body,
        out_type=jax.ShapeDtypeStruct((N_COLS,), jnp.int32),
        mesh=plsc.VectorSubcoreMesh(core_axis_name="c", subcore_axis_name="s"),
        scratch_types=[
            pltpu.VMEM((N_VOTERS, CHUNK), jnp.int32),
            pltpu.VMEM((CHUNK,), jnp.int32),
        ],
        compiler_params=pltpu.CompilerParams(use_tc_tiling_on_sc=False),
    )
    return k(inputs)


def kernel(inputs):
    return _vote(inputs)


# speedup vs baseline: 20.8531x; 20.8531x over previous
import jax
import jax.numpy as jnp
from jax import lax
from jax.experimental import pallas as pl
from jax.experimental.pallas import tpu as pltpu
from jax.experimental.pallas import tpu_sc as plsc

N_VOTERS = 32
N_COLS = 1_000_000
LANES = 16
NUM_WORKERS = 32
CHUNK = 2048
FULL_CHUNKS = N_COLS // CHUNK
TAIL = N_COLS - FULL_CHUNKS * CHUNK
TAIL_WORKER = FULL_CHUNKS % NUM_WORKERS
MAX_K = -(-FULL_CHUNKS // NUM_WORKERS)


def _body(in_hbm, tail_hbm, out_hbm, buf, tail_buf, out_buf):
    c = lax.axis_index("c")
    s = lax.axis_index("s")
    wid = s * 2 + c

    def reduce_cols(src, n_cols):
        def col_group(j, _):
            off = j * LANES
            acc = src[0, pl.ds(off, LANES)]
            for i in range(1, N_VOTERS):
                acc = acc + src[i, pl.ds(off, LANES)]
            out_buf[pl.ds(off, LANES)] = jnp.where(
                acc > N_VOTERS // 2, 1, 0
            ).astype(jnp.int32)
            return 0

        lax.fori_loop(0, n_cols // LANES, col_group, 0)

    def do_chunk(chunk_id):
        base = chunk_id * CHUNK
        pltpu.sync_copy(in_hbm.at[:, pl.ds(base, CHUNK)], buf)
        reduce_cols(buf, CHUNK)
        pltpu.sync_copy(out_buf, out_hbm.at[pl.ds(base, CHUNK)])

    for k in range(MAX_K):
        chunk_id = wid + k * NUM_WORKERS

        @pl.when(chunk_id < FULL_CHUNKS)
        def _():
            do_chunk(chunk_id)

    @pl.when(wid == TAIL_WORKER)
    def _():
        base = FULL_CHUNKS * CHUNK
        pltpu.sync_copy(tail_hbm, tail_buf)
        reduce_cols(tail_buf, TAIL)
        pltpu.sync_copy(
            out_buf.at[pl.ds(0, TAIL)], out_hbm.at[pl.ds(base, TAIL)]
        )


@jax.jit
def _vote(inputs):
    tail = lax.slice(inputs, (0, FULL_CHUNKS * CHUNK), (N_VOTERS, N_COLS))
    k = pl.kernel(
        _body,
        out_type=jax.ShapeDtypeStruct((N_COLS,), jnp.int32),
        mesh=plsc.VectorSubcoreMesh(core_axis_name="c", subcore_axis_name="s"),
        scratch_types=[
            pltpu.VMEM((N_VOTERS, CHUNK), jnp.int32),
            pltpu.VMEM((N_VOTERS, TAIL), jnp.int32),
            pltpu.VMEM((CHUNK,), jnp.int32),
        ],
    )
    return k(inputs, tail)


def kernel(inputs):
    return _vote(inputs)


# speedup vs baseline: 31.9155x; 1.5305x over previous
import jax
import jax.numpy as jnp
from jax import lax
from jax.experimental import pallas as pl
from jax.experimental.pallas import tpu as pltpu
from jax.experimental.pallas import tpu_sc as plsc

N_VOTERS = 32
N_COLS = 1_000_000
LANES = 16
NUM_WORKERS = 32
PER_WORKER = 31232
TAIL = N_COLS - NUM_WORKERS * PER_WORKER
TAIL_PAD = 640
TAIL_WORKER = 0
CHUNK = 1536
CHUNK_SIZES = [CHUNK] * (PER_WORKER // CHUNK) + [PER_WORKER % CHUNK]
NCHUNKS = len(CHUNK_SIZES)
CHUNK_OFFS = [i * CHUNK for i in range(NCHUNKS)]


def _body(in_hbm, tail_hbm, out_hbm, buf0, buf1, out_acc, isem0, isem1, osem):
    c = lax.axis_index("c")
    s = lax.axis_index("s")
    wid = s * 2 + c
    base = wid * PER_WORKER
    bufs = (buf0, buf1)
    isems = (isem0, isem1)

    def start_in(k):
        pltpu.async_copy(
            in_hbm.at[:, pl.ds(base + CHUNK_OFFS[k], CHUNK_SIZES[k])],
            bufs[k % 2].at[:, pl.ds(0, CHUNK_SIZES[k])],
            isems[k % 2],
        )

    def reduce_cols(src, out_off, n_cols):
        def col_group(j, _):
            off = j * LANES
            acc = src[0, pl.ds(off, LANES)]
            for i in range(1, N_VOTERS):
                acc = acc + src[i, pl.ds(off, LANES)]
            out_acc[pl.ds(out_off + off, LANES)] = jnp.where(
                acc > N_VOTERS // 2, 1, 0
            ).astype(jnp.int32)
            return 0

        lax.fori_loop(0, n_cols // LANES, col_group, 0)

    start_in(0)
    for k in range(NCHUNKS):
        if k + 1 < NCHUNKS:
            start_in(k + 1)
        pltpu.make_async_copy(
            in_hbm.at[:, pl.ds(base + CHUNK_OFFS[k], CHUNK_SIZES[k])],
            bufs[k % 2].at[:, pl.ds(0, CHUNK_SIZES[k])],
            isems[k % 2],
        ).wait()
        reduce_cols(bufs[k % 2], CHUNK_OFFS[k], CHUNK_SIZES[k])

    pltpu.async_copy(out_acc, out_hbm.at[pl.ds(base, PER_WORKER)], osem)
    pltpu.make_async_copy(
        out_acc, out_hbm.at[pl.ds(base, PER_WORKER)], osem
    ).wait()

    @pl.when(wid == TAIL_WORKER)
    def _():
        tbase = NUM_WORKERS * PER_WORKER
        pltpu.sync_copy(tail_hbm, buf0.at[:, pl.ds(0, TAIL_PAD)])
        reduce_cols(buf0, 0, TAIL)
        pltpu.sync_copy(
            out_acc.at[pl.ds(0, TAIL)], out_hbm.at[pl.ds(tbase, TAIL)]
        )


@jax.jit
def _vote(inputs):
    tail = lax.slice(
        inputs, (0, NUM_WORKERS * PER_WORKER), (N_VOTERS, N_COLS)
    )
    tail = jnp.pad(tail, ((0, 0), (0, TAIL_PAD - TAIL)))
    k = pl.kernel(
        _body,
        out_type=jax.ShapeDtypeStruct((N_COLS,), jnp.int32),
        mesh=plsc.VectorSubcoreMesh(core_axis_name="c", subcore_axis_name="s"),
        scratch_types=[
            pltpu.VMEM((N_VOTERS, CHUNK), jnp.int32),
            pltpu.VMEM((N_VOTERS, CHUNK), jnp.int32),
            pltpu.VMEM((PER_WORKER,), jnp.int32),
            pltpu.SemaphoreType.DMA,
            pltpu.SemaphoreType.DMA,
            pltpu.SemaphoreType.DMA,
        ],
    )
    return k(inputs, tail)


def kernel(inputs):
    return _vote(inputs)


# speedup vs baseline: 33.0468x; 1.0354x over previous
import jax
import jax.numpy as jnp
from jax import lax
from jax.experimental import pallas as pl
from jax.experimental.pallas import tpu as pltpu
from jax.experimental.pallas import tpu_sc as plsc

N_VOTERS = 32
N_COLS = 1_000_000
LANES = 16
NUM_WORKERS = 32
PER_WORKER = 31232
TAIL = N_COLS - NUM_WORKERS * PER_WORKER
TAIL_PAD = 640
TAIL_WORKER = 0
CHUNK = 1536
CHUNK_SIZES = [CHUNK] * (PER_WORKER // CHUNK) + [PER_WORKER % CHUNK]
NCHUNKS = len(CHUNK_SIZES)
CHUNK_OFFS = [i * CHUNK for i in range(NCHUNKS)]


def _body(in_hbm, tail_hbm, out_hbm, buf0, buf1, out_acc, isem0, isem1, osem):
    c = lax.axis_index("c")
    s = lax.axis_index("s")
    wid = s * 2 + c
    base = wid * PER_WORKER
    bufs = (buf0, buf1)
    isems = (isem0, isem1)

    def start_in(k):
        pltpu.async_copy(
            in_hbm.at[:, pl.ds(base + CHUNK_OFFS[k], CHUNK_SIZES[k])],
            bufs[k % 2].at[:, pl.ds(0, CHUNK_SIZES[k])],
            isems[k % 2],
        )

    def reduce_cols(src, out_off, n_cols):
        @plsc.parallel_loop(0, n_cols // LANES, unroll=2)
        def col_group(j):
            off = j * LANES
            vals = [src[i, pl.ds(off, LANES)] for i in range(N_VOTERS)]
            while len(vals) > 1:
                vals = [
                    vals[i] + vals[i + 1] for i in range(0, len(vals), 2)
                ]
            out_acc[pl.ds(out_off + off, LANES)] = jnp.where(
                vals[0] > N_VOTERS // 2, 1, 0
            ).astype(jnp.int32)

    start_in(0)
    for k in range(NCHUNKS):
        if k + 1 < NCHUNKS:
            start_in(k + 1)
        pltpu.make_async_copy(
            in_hbm.at[:, pl.ds(base + CHUNK_OFFS[k], CHUNK_SIZES[k])],
            bufs[k % 2].at[:, pl.ds(0, CHUNK_SIZES[k])],
            isems[k % 2],
        ).wait()
        reduce_cols(bufs[k % 2], CHUNK_OFFS[k], CHUNK_SIZES[k])

    pltpu.async_copy(out_acc, out_hbm.at[pl.ds(base, PER_WORKER)], osem)
    pltpu.make_async_copy(
        out_acc, out_hbm.at[pl.ds(base, PER_WORKER)], osem
    ).wait()

    @pl.when(wid == TAIL_WORKER)
    def _():
        tbase = NUM_WORKERS * PER_WORKER
        pltpu.sync_copy(tail_hbm, buf0.at[:, pl.ds(0, TAIL_PAD)])
        reduce_cols(buf0, 0, TAIL)
        pltpu.sync_copy(
            out_acc.at[pl.ds(0, TAIL)], out_hbm.at[pl.ds(tbase, TAIL)]
        )


@jax.jit
def _vote(inputs):
    tail = lax.slice(
        inputs, (0, NUM_WORKERS * PER_WORKER), (N_VOTERS, N_COLS)
    )
    tail = jnp.pad(tail, ((0, 0), (0, TAIL_PAD - TAIL)))
    k = pl.kernel(
        _body,
        out_type=jax.ShapeDtypeStruct((N_COLS,), jnp.int32),
        mesh=plsc.VectorSubcoreMesh(core_axis_name="c", subcore_axis_name="s"),
        scratch_types=[
            pltpu.VMEM((N_VOTERS, CHUNK), jnp.int32),
            pltpu.VMEM((N_VOTERS, CHUNK), jnp.int32),
            pltpu.VMEM((PER_WORKER,), jnp.int32),
            pltpu.SemaphoreType.DMA,
            pltpu.SemaphoreType.DMA,
            pltpu.SemaphoreType.DMA,
        ],
    )
    return k(inputs, tail)


def kernel(inputs):
    return _vote(inputs)
